# Initial kernel scaffold; baseline (speedup 1.0000x reference)
import jax
import jax.numpy as jnp
from jax.experimental import pallas as pl


def kernel(items, params):
    raise NotImplementedError("write your pallas kernel here")


# speedup vs baseline: 485.2323x; 485.2323x over previous
import functools

import jax
import jax.numpy as jnp
from jax import lax
from jax.experimental import pallas as pl
from jax.experimental.pallas import tpu as pltpu
from jax.experimental.pallas import tpu_sc as plsc

L = 50000
D = 128
DH = 16
NW = 32
LP = 50176
BPW = LP // NW
CH = 112
T = 1024
G = LP // T


def _sc_gather(table, idx):
    mesh = plsc.VectorSubcoreMesh(core_axis_name="c", subcore_axis_name="s")

    @functools.partial(
        pl.kernel,
        out_type=jax.ShapeDtypeStruct((LP, D), jnp.float32),
        mesh=mesh,
        scratch_types=[
            pltpu.VMEM((BPW,), jnp.int32),
            pltpu.VMEM((CH, D), jnp.float32),
            pltpu.SemaphoreType.DMA,
        ],
    )
    def k(table_hbm, idx_hbm, out_hbm, idx_v, rows_v, sem):
        wid = lax.axis_index("s") * 2 + lax.axis_index("c")
        base = wid * BPW
        pltpu.sync_copy(idx_hbm.at[pl.ds(base, BPW)], idx_v)
        for c in range(BPW // CH):
            pltpu.async_copy(
                table_hbm.at[idx_v.at[pl.ds(c * CH, CH)]], rows_v, sem
            ).wait()
            pltpu.sync_copy(rows_v, out_hbm.at[pl.ds(base + c * CH, CH)])

    return k(table, idx)


def _shift1(x, carry):
    r = pltpu.roll(x, shift=1, axis=0)
    li = lax.broadcasted_iota(jnp.int32, x.shape, 0)
    return jnp.where(li == 0, carry, r)


def _dense_body(hs_ref, wsq_ref, bones_ref, alf_ref,
                w11_ref, w12_ref, w13_ref, w21_ref, w22_ref, w23_ref,
                bs1_ref, bs2_ref, bs3_ref,
                b11_ref, b12_ref, b13_ref, b21_ref, b22_ref, b23_ref,
                gwp_ref, out_ref, acc1, acc2, cp, cq, ce):
    t = pl.program_id(0)

    @pl.when(t == 0)
    def _init():
        acc1[...] = jnp.zeros_like(acc1)
        acc2[...] = jnp.zeros_like(acc2)
        cp[...] = jnp.zeros_like(cp)
        cq[...] = jnp.zeros_like(cq)
        ce[...] = jnp.zeros_like(ce)

    hs = hs_ref[...]
    pq = jnp.dot(hs, wsq_ref[...], preferred_element_type=jnp.float32)
    p = pq[:, :D]
    q = pq[:, D:]
    e_full = jnp.dot(q * alf_ref[...], bones_ref[...],
                     preferred_element_type=jnp.float32)
    el = jnp.where(e_full > 0, e_full, 0.2 * e_full)

    p_prev = _shift1(p, cp[...])
    q_prev = _shift1(q, cq[...])
    el_prev = _shift1(el, ce[...])
    cp[...] = p[-1:, :]
    cq[...] = q[-1:, :]
    ce[...] = el[-1:, :]

    row = t * T + lax.broadcasted_iota(jnp.int32, (T, 1), 0)

    x1 = hs + bs1_ref[...] + p_prev
    r1 = jnp.maximum(
        jnp.dot(x1, w11_ref[...], preferred_element_type=jnp.float32)
        + b11_ref[...], 0.0)
    r1 = jnp.where(row < L, r1, 0.0)
    acc1[...] += jnp.sum(r1, axis=0, keepdims=True)

    a = 1.0 / (1.0 + jnp.exp(el - el_prev))
    msg = a * q_prev + (1.0 - a) * q
    x2 = msg + bs2_ref[...]
    r2 = jnp.maximum(
        jnp.dot(x2, w12_ref[...], preferred_element_type=jnp.float32)
        + b12_ref[...], 0.0)
    r2 = jnp.where((row >= 1) & (row <= L - 1), r2, 0.0)
    acc2[...] += jnp.sum(r2, axis=0, keepdims=True)

    @pl.when(t == G - 1)
    def _final():
        m1 = acc1[...] * (1.0 / L)
        m2 = acc2[...] * (1.0 / (L - 1))
        rep1 = jnp.dot(m1, w21_ref[...],
                       preferred_element_type=jnp.float32) + b21_ref[...]
        rep2 = jnp.dot(m2, w22_ref[...],
                       preferred_element_type=jnp.float32) + b22_ref[...]
        h3 = jnp.maximum(
            jnp.dot(bs3_ref[...], w13_ref[...],
                    preferred_element_type=jnp.float32) + b13_ref[...], 0.0)
        rep3 = jnp.dot(h3, w23_ref[...],
                       preferred_element_type=jnp.float32) + b23_ref[...]
        gwp = gwp_ref[...]
        ew = jnp.exp(gwp - jnp.max(gwp))
        wv = ew / jnp.sum(ew)
        fused = (wv[:, 0:1] * rep1 + wv[:, 1:2] * rep2 + wv[:, 2:3] * rep3)
        out_ref[...] = jnp.zeros((8, D), jnp.float32)
        out_ref[0:1, :] = rep1
        out_ref[1:2, :] = rep2
        out_ref[2:3, :] = rep3
        out_ref[3:4, :] = fused
        out_ref[4:5, :] = wv


def _dense(hs1p, wsq, bones, alf, ws, bs):
    full = lambda shape: pl.BlockSpec(shape, lambda t: (0, 0))
    v = full((1, D))
    return pl.pallas_call(
        _dense_body,
        grid=(G,),
        in_specs=[
            pl.BlockSpec((T, D), lambda t: (t, 0)),
            full((D, 2 * D)),
            full((D, D)),
            v,
            full((D, D)), full((D, D)), full((D, D)),
            full((D, D)), full((D, D)), full((D, D)),
            v, v, v,
            v, v, v,
            v, v, v,
            v,
        ],
        out_specs=pl.BlockSpec((8, D), lambda t: (0, 0)),
        out_shape=jax.ShapeDtypeStruct((8, D), jnp.float32),
        scratch_shapes=[pltpu.VMEM((1, D), jnp.float32)] * 5,
        compiler_params=pltpu.CompilerParams(
            dimension_semantics=("arbitrary",)),
    )(hs1p, wsq, bones, alf, *ws, *bs)


def kernel(items, params):
    idxp = jnp.pad(items, (0, LP - L))
    hs1p = _sc_gather(params['emb'], idxp)

    wsq = jnp.concatenate([params['W_seq1'], params['W_up1']], axis=1)
    gidx = jnp.arange(D, dtype=jnp.int32) // DH
    bones = (gidx[:, None] == gidx[None, :]).astype(jnp.float32)
    alf = params['al_up1'].reshape(1, D)
    ws = (params['ro1_W1'], params['ro2_W1'], params['ro3_W1'],
          params['ro1_W2'], params['ro2_W2'], params['ro3_W2'])
    r = lambda x: x.reshape(1, D)
    bs = (r(params['b_seq1'] + params['b_down1']),
          r(params['b_seq2'] + params['b_down2'] + params['b_up1']),
          r(params['b_seq3'] + params['b_up2']),
          r(params['ro1_b1']), r(params['ro2_b1']), r(params['ro3_b1']),
          r(params['ro1_b2']), r(params['ro2_b2']), r(params['ro3_b2']),
          jnp.concatenate([params['gw'],
                           jnp.full((D - 3,), -jnp.inf, jnp.float32)]
                          ).reshape(1, D))

    res = _dense(hs1p, wsq, bones, alf, ws, bs)
    return res[3], res[0:3], res[4, :3]


# speedup vs baseline: 515.6375x; 1.0627x over previous
import functools

import jax
import jax.numpy as jnp
from jax import lax
from jax.experimental import pallas as pl
from jax.experimental.pallas import tpu as pltpu
from jax.experimental.pallas import tpu_sc as plsc

L = 50000
D = 128
DH = 16
NW = 32
LP = 50176
BPW = LP // NW
CH = 112
CB = 2 * CH
T = 1024
G = LP // T


def _sc_gather(table, idx):
    mesh = plsc.VectorSubcoreMesh(core_axis_name="c", subcore_axis_name="s")

    @functools.partial(
        pl.kernel,
        out_type=jax.ShapeDtypeStruct((LP, D), jnp.float32),
        mesh=mesh,
        scratch_types=[
            pltpu.VMEM((BPW,), jnp.int32),
            pltpu.VMEM((CB, D), jnp.float32),
            pltpu.VMEM((CB, D), jnp.float32),
            pltpu.SemaphoreType.DMA,
            pltpu.SemaphoreType.DMA,
        ],
    )
    def k(table_hbm, idx_hbm, out_hbm, idx_v, rows0, rows1, sem0, sem1):
        wid = lax.axis_index("s") * 2 + lax.axis_index("c")
        base = wid * BPW
        pltpu.sync_copy(idx_hbm.at[pl.ds(base, BPW)], idx_v)
        rows = (rows0, rows1)
        sems = (sem0, sem1)
        nsteps = BPW // CB

        def fire(step, buf):
            hnds = []
            for h in range(2):
                hnds.append(pltpu.async_copy(
                    table_hbm.at[idx_v.at[pl.ds(step * CB + h * CH, CH)]],
                    rows[buf].at[pl.ds(h * CH, CH)],
                    sems[buf]))
            return hnds

        pend = fire(0, 0)
        for c in range(nsteps):
            b = c & 1
            if c + 1 < nsteps:
                nxt = fire(c + 1, (c + 1) & 1)
            for h in pend:
                h.wait()
            pltpu.sync_copy(rows[b], out_hbm.at[pl.ds(base + c * CB, CB)])
            if c + 1 < nsteps:
                pend = nxt

    return k(table, idx)


def _shift1(x, carry):
    r = pltpu.roll(x, shift=1, axis=0)
    li = lax.broadcasted_iota(jnp.int32, x.shape, 0)
    return jnp.where(li == 0, carry, r)


def _dense_body(hs_ref, wprev_ref, wcur_ref,
                w11_ref, w12_ref, w13_ref, w21_ref, w22_ref, w23_ref,
                bs1_ref, bs2_ref, bs3_ref,
                b11_ref, b12_ref, b13_ref, b21_ref, b22_ref, b23_ref,
                gwp_ref, out_ref, acc1, acc2, ch):
    t = pl.program_id(0)

    @pl.when(t == 0)
    def _init():
        acc1[...] = jnp.zeros_like(acc1)
        acc2[...] = jnp.zeros_like(acc2)
        ch[...] = jnp.zeros_like(ch)

    hs = hs_ref[...]
    hs_prev = _shift1(hs, ch[...])
    ch[...] = hs[-1:, :]

    ppe = jnp.dot(hs_prev, wprev_ref[...], preferred_element_type=jnp.float32)
    p_prev = ppe[:, :D]
    q_prev = ppe[:, D:2 * D]
    e_prev = ppe[:, 2 * D:]
    qe = jnp.dot(hs, wcur_ref[...], preferred_element_type=jnp.float32)
    q = qe[:, :D]
    e_cur = qe[:, D:]
    el = jnp.maximum(e_cur, 0.2 * e_cur)
    el_prev = jnp.maximum(e_prev, 0.2 * e_prev)

    x1 = hs + bs1_ref[...] + p_prev
    r1 = jnp.maximum(
        jnp.dot(x1, w11_ref[...], preferred_element_type=jnp.float32)
        + b11_ref[...], 0.0)

    a = 1.0 / (1.0 + jnp.exp(el - el_prev))
    msg = q + a * (q_prev - q)
    x2 = msg + bs2_ref[...]
    r2 = jnp.maximum(
        jnp.dot(x2, w12_ref[...], preferred_element_type=jnp.float32)
        + b12_ref[...], 0.0)

    @pl.when((t > 0) & (t < G - 1))
    def _mid():
        acc1[...] += jnp.sum(r1, axis=0, keepdims=True)
        acc2[...] += jnp.sum(r2, axis=0, keepdims=True)

    @pl.when((t == 0) | (t == G - 1))
    def _edge():
        row = t * T + lax.broadcasted_iota(jnp.int32, (T, 1), 0)
        r1m = jnp.where(row < L, r1, 0.0)
        r2m = jnp.where((row >= 1) & (row <= L - 1), r2, 0.0)
        acc1[...] += jnp.sum(r1m, axis=0, keepdims=True)
        acc2[...] += jnp.sum(r2m, axis=0, keepdims=True)

    @pl.when(t == G - 1)
    def _final():
        m1 = acc1[...] * (1.0 / L)
        m2 = acc2[...] * (1.0 / (L - 1))
        rep1 = jnp.dot(m1, w21_ref[...],
                       preferred_element_type=jnp.float32) + b21_ref[...]
        rep2 = jnp.dot(m2, w22_ref[...],
                       preferred_element_type=jnp.float32) + b22_ref[...]
        h3 = jnp.maximum(
            jnp.dot(bs3_ref[...], w13_ref[...],
                    preferred_element_type=jnp.float32) + b13_ref[...], 0.0)
        rep3 = jnp.dot(h3, w23_ref[...],
                       preferred_element_type=jnp.float32) + b23_ref[...]
        gwp = gwp_ref[...]
        ew = jnp.exp(gwp - jnp.max(gwp))
        wv = ew / jnp.sum(ew)
        fused = (wv[:, 0:1] * rep1 + wv[:, 1:2] * rep2 + wv[:, 2:3] * rep3)
        out_ref[...] = jnp.zeros((8, D), jnp.float32)
        out_ref[0:1, :] = rep1
        out_ref[1:2, :] = rep2
        out_ref[2:3, :] = rep3
        out_ref[3:4, :] = fused
        out_ref[4:5, :] = wv


def _dense(hs1p, wprev, wcur, ws, bs):
    full = lambda shape: pl.BlockSpec(shape, lambda t: (0, 0))
    v = full((1, D))
    return pl.pallas_call(
        _dense_body,
        grid=(G,),
        in_specs=[
            pl.BlockSpec((T, D), lambda t: (t, 0)),
            full((D, 3 * D)),
            full((D, 2 * D)),
            full((D, D)), full((D, D)), full((D, D)),
            full((D, D)), full((D, D)), full((D, D)),
            v, v, v,
            v, v, v,
            v, v, v,
            v,
        ],
        out_specs=pl.BlockSpec((8, D), lambda t: (0, 0)),
        out_shape=jax.ShapeDtypeStruct((8, D), jnp.float32),
        scratch_shapes=[pltpu.VMEM((1, D), jnp.float32)] * 3,
        compiler_params=pltpu.CompilerParams(
            dimension_semantics=("arbitrary",)),
    )(hs1p, wprev, wcur, *ws, *bs)


def kernel(items, params):
    idxp = jnp.pad(items, (0, LP - L))
    hs1p = _sc_gather(params['emb'], idxp)

    gidx = jnp.arange(D, dtype=jnp.int32) // DH
    bones = (gidx[:, None] == gidx[None, :]).astype(jnp.float32)
    wel = jnp.dot(params['W_up1'] * params['al_up1'].reshape(1, D), bones)
    wprev = jnp.concatenate([params['W_seq1'], params['W_up1'], wel], axis=1)
    wcur = jnp.concatenate([params['W_up1'], wel], axis=1)
    ws = (params['ro1_W1'], params['ro2_W1'], params['ro3_W1'],
          params['ro1_W2'], params['ro2_W2'], params['ro3_W2'])
    r = lambda x: x.reshape(1, D)
    bs = (r(params['b_seq1'] + params['b_down1']),
          r(params['b_seq2'] + params['b_down2'] + params['b_up1']),
          r(params['b_seq3'] + params['b_up2']),
          r(params['ro1_b1']), r(params['ro2_b1']), r(params['ro3_b1']),
          r(params['ro1_b2']), r(params['ro2_b2']), r(params['ro3_b2']),
          jnp.concatenate([params['gw'],
                           jnp.full((D - 3,), -jnp.inf, jnp.float32)]
                          ).reshape(1, D))

    res = _dense(hs1p, wprev, wcur, ws, bs)
    return res[3], res[0:3], res[4, :3]


# speedup vs baseline: 577.2988x; 1.1196x over previous
import functools

import jax
import jax.numpy as jnp
from jax import lax
from jax.experimental import pallas as pl
from jax.experimental.pallas import tpu as pltpu
from jax.experimental.pallas import tpu_sc as plsc

L = 50000
D = 128
DH = 16
NW = 32
LP = 50176
CH = 112
CB = 2 * CH
T = 1024
G = LP // T
PHASES = (7, 14, 14, 14)


def _sc_gather(table, idx, offt, nt):
    mesh = plsc.VectorSubcoreMesh(core_axis_name="c", subcore_axis_name="s")
    bpw = nt * T // NW
    nsteps = bpw // CB
    off = offt * T

    @functools.partial(
        pl.kernel,
        out_type=jax.ShapeDtypeStruct((nt * T, D), jnp.float32),
        mesh=mesh,
        scratch_types=[
            pltpu.VMEM((bpw,), jnp.int32),
            pltpu.VMEM((CB, D), jnp.float32),
            pltpu.VMEM((CB, D), jnp.float32),
            pltpu.SemaphoreType.DMA,
            pltpu.SemaphoreType.DMA,
            pltpu.SemaphoreType.DMA,
            pltpu.SemaphoreType.DMA,
        ],
    )
    def k(table_hbm, idx_hbm, out_hbm, idx_v, rows0, rows1,
          sem0, sem1, osem0, osem1):
        wid = lax.axis_index("s") * 2 + lax.axis_index("c")
        base = wid * bpw
        pltpu.sync_copy(idx_hbm.at[pl.ds(off + base, bpw)], idx_v)
        rows = (rows0, rows1)
        sems = (sem0, sem1)
        osems = (osem0, osem1)

        def fire(step, buf):
            hnds = []
            for h in range(2):
                hnds.append(pltpu.async_copy(
                    table_hbm.at[idx_v.at[pl.ds(step * CB + h * CH, CH)]],
                    rows[buf].at[pl.ds(h * CH, CH)],
                    sems[buf]))
            return hnds

        pend_g = fire(0, 0)
        pend_o = [None, None]
        for c in range(nsteps):
            b = c & 1
            nb = (c + 1) & 1
            if c + 1 < nsteps:
                if pend_o[nb] is not None:
                    pend_o[nb].wait()
                    pend_o[nb] = None
                nxt = fire(c + 1, nb)
            for h in pend_g:
                h.wait()
            pend_o[b] = pltpu.async_copy(
                rows[b], out_hbm.at[pl.ds(base + c * CB, CB)], osems[b])
            if c + 1 < nsteps:
                pend_g = nxt
        for b in range(2):
            if pend_o[b] is not None:
                pend_o[b].wait()

    return k(table, idx)


def _shift1(x, carry):
    r = pltpu.roll(x, shift=1, axis=0)
    li = lax.broadcasted_iota(jnp.int32, x.shape, 0)
    return jnp.where(li == 0, carry, r)


def _make_phase_body(offt, nt, first, last):
    def body(hs_ref, accin_ref, chin_ref, wprev_ref, wcur_ref,
             w1212_ref, b1212_ref, w13_ref, w21_ref, w22_ref, w23_ref,
             bs3_ref, b13_ref, b21_ref, b22_ref, b23_ref, gwp_ref, *rest):
        if last:
            accout_ref, chout_ref, of_ref, os_ref, ow_ref, acc12, ch = rest
        else:
            accout_ref, chout_ref, acc12, ch = rest
        t = pl.program_id(0)

        @pl.when(t == 0)
        def _init():
            acc12[...] = accin_ref[...]
            ch[...] = chin_ref[...]

        hs = hs_ref[...]
        hs_prev = _shift1(hs, ch[...])
        ch[...] = hs[-1:, :]

        ppe = jnp.dot(hs_prev, wprev_ref[...],
                      preferred_element_type=jnp.float32)
        p_prev = ppe[:, :D]
        q_prev = ppe[:, D:2 * D]
        e_prev = ppe[:, 2 * D:]
        qe = jnp.dot(hs, wcur_ref[...], preferred_element_type=jnp.float32)
        q = qe[:, :D]
        e_cur = qe[:, D:]
        el = jnp.maximum(e_cur, 0.2 * e_cur)
        el_prev = jnp.maximum(e_prev, 0.2 * e_prev)

        a = 1.0 / (1.0 + jnp.exp(el - el_prev))
        msg = q + a * (q_prev - q)

        rin = jnp.concatenate([hs + p_prev, msg], axis=1)
        r12 = jnp.maximum(
            jnp.dot(rin, w1212_ref[...], preferred_element_type=jnp.float32)
            + b1212_ref[...], 0.0)

        edge = None
        if first and nt == 1:
            edge = (t == 0)
        elif first:
            edge = (t == 0)
        if last:
            e2 = (t == nt - 1)
            edge = e2 if edge is None else (edge | e2)

        if edge is None:
            acc12[...] += jnp.sum(r12, axis=0, keepdims=True)
        else:
            @pl.when(jnp.logical_not(edge))
            def _mid():
                acc12[...] += jnp.sum(r12, axis=0, keepdims=True)

            @pl.when(edge)
            def _edge():
                row = (offt + t) * T + lax.broadcasted_iota(
                    jnp.int32, (T, 1), 0)
                r1m = jnp.where(row < L, r12[:, :D], 0.0)
                r2m = jnp.where((row >= 1) & (row <= L - 1), r12[:, D:], 0.0)
                acc12[:, :D] += jnp.sum(r1m, axis=0, keepdims=True)
                acc12[:, D:] += jnp.sum(r2m, axis=0, keepdims=True)

        @pl.when(t == nt - 1)
        def _fin():
            accout_ref[...] = acc12[...]
            chout_ref[...] = ch[...]
            if last:
                m1 = acc12[:, :D] * (1.0 / L)
                m2 = acc12[:, D:] * (1.0 / (L - 1))
                rep1 = jnp.dot(m1, w21_ref[...],
                               preferred_element_type=jnp.float32) + b21_ref[...]
                rep2 = jnp.dot(m2, w22_ref[...],
                               preferred_element_type=jnp.float32) + b22_ref[...]
                h3 = jnp.maximum(
                    jnp.dot(bs3_ref[...], w13_ref[...],
                            preferred_element_type=jnp.float32)
                    + b13_ref[...], 0.0)
                rep3 = jnp.dot(h3, w23_ref[...],
                               preferred_element_type=jnp.float32) + b23_ref[...]
                gwp = gwp_ref[...]
                ew = jnp.exp(gwp - jnp.max(gwp))
                wv = ew / jnp.sum(ew)
                fused = (wv[:, 0:1] * rep1 + wv[:, 1:2] * rep2
                         + wv[:, 2:3] * rep3)
                of_ref[...] = fused
                os_ref[0:1, :] = rep1
                os_ref[1:2, :] = rep2
                os_ref[2:3, :] = rep3
                ow_ref[...] = wv

    return body


def _dense_phase(hs_k, acc, chv, consts, offt, nt, first, last):
    full = lambda shape: pl.BlockSpec(shape, lambda t: (0, 0))
    v = full((1, D))
    out_specs = [full((1, 2 * D)), v]
    out_shape = [jax.ShapeDtypeStruct((1, 2 * D), jnp.float32),
                 jax.ShapeDtypeStruct((1, D), jnp.float32)]
    if last:
        out_specs += [v, full((3, D)), v]
        out_shape += [jax.ShapeDtypeStruct((1, D), jnp.float32),
                      jax.ShapeDtypeStruct((3, D), jnp.float32),
                      jax.ShapeDtypeStruct((1, D), jnp.float32)]
    return pl.pallas_call(
        _make_phase_body(offt, nt, first, last),
        grid=(nt,),
        in_specs=[
            pl.BlockSpec((T, D), lambda t: (t, 0)),
            full((1, 2 * D)),
            v,
            full((D, 3 * D)),
            full((D, 2 * D)),
            full((2 * D, 2 * D)),
            full((1, 2 * D)),
            full((D, D)),
            full((D, D)), full((D, D)), full((D, D)),
            v,
            v,
            v, v, v,
            v,
        ],
        out_specs=out_specs,
        out_shape=out_shape,
        scratch_shapes=[pltpu.VMEM((1, 2 * D), jnp.float32),
                        pltpu.VMEM((1, D), jnp.float32)],
        compiler_params=pltpu.CompilerParams(
            dimension_semantics=("arbitrary",)),
    )(hs_k, acc, chv, *consts)


def kernel(items, params):
    idxp = jnp.pad(items, (0, LP - L))

    gidx = jnp.arange(D, dtype=jnp.int32) // DH
    bones = (gidx[:, None] == gidx[None, :]).astype(jnp.float32)
    wel = jnp.dot(params['W_up1'] * params['al_up1'].reshape(1, D), bones)
    wprev = jnp.concatenate([params['W_seq1'], params['W_up1'], wel], axis=1)
    wcur = jnp.concatenate([params['W_up1'], wel], axis=1)
    z = jnp.zeros((D, D), jnp.float32)
    w1212 = jnp.block([[params['ro1_W1'], z], [z, params['ro2_W1']]])
    bsum1 = params['b_seq1'] + params['b_down1']
    bsum2 = params['b_seq2'] + params['b_down2'] + params['b_up1']
    b1212 = jnp.concatenate(
        [jnp.dot(bsum1, params['ro1_W1']) + params['ro1_b1'],
         jnp.dot(bsum2, params['ro2_W1']) + params['ro2_b1']]).reshape(1, 2 * D)
    r = lambda x: x.reshape(1, D)
    consts = (wprev, wcur, w1212, b1212,
              params['ro3_W1'],
              params['ro1_W2'], params['ro2_W2'], params['ro3_W2'],
              r(params['b_seq3'] + params['b_up2']),
              r(params['ro3_b1']),
              r(params['ro1_b2']), r(params['ro2_b2']), r(params['ro3_b2']),
              jnp.concatenate([params['gw'],
                               jnp.full((D - 3,), -jnp.inf, jnp.float32)]
                              ).reshape(1, D))

    segs = []
    offt = 0
    for nt in PHASES:
        segs.append((offt, nt))
        offt += nt
    hs_parts = [_sc_gather(params['emb'], idxp, o, n) for o, n in segs]

    acc = jnp.zeros((1, 2 * D), jnp.float32)
    chv = jnp.zeros((1, D), jnp.float32)
    for k_i, (o, n) in enumerate(segs):
        res = _dense_phase(hs_parts[k_i], acc, chv, consts, o, n,
                           first=(k_i == 0), last=(k_i == len(segs) - 1))
        acc, chv = res[0], res[1]
    of, os_, ow = res[2], res[3], res[4]
    return of.reshape(D), os_, ow[0, :3]


# speedup vs baseline: 583.5342x; 1.0108x over previous
import functools

import jax
import jax.numpy as jnp
from jax import lax
from jax.experimental import pallas as pl
from jax.experimental.pallas import tpu as pltpu
from jax.experimental.pallas import tpu_sc as plsc

L = 50000
D = 128
DH = 16
NW = 32
LP = 50176
CH = 112
CB = 2 * CH
T = 1024
G = LP // T
PHASES = (7, 14, 14, 14)


def _sc_gather(table, idx, offt, nt):
    mesh = plsc.VectorSubcoreMesh(core_axis_name="c", subcore_axis_name="s")
    bpw = nt * T // NW
    nsteps = bpw // CB
    off = offt * T

    @functools.partial(
        pl.kernel,
        out_type=jax.ShapeDtypeStruct((nt * T, D), jnp.float32),
        mesh=mesh,
        scratch_types=[
            pltpu.VMEM((bpw,), jnp.int32),
            pltpu.VMEM((CB, D), jnp.float32),
            pltpu.VMEM((CB, D), jnp.float32),
            pltpu.SemaphoreType.DMA,
            pltpu.SemaphoreType.DMA,
            pltpu.SemaphoreType.DMA,
            pltpu.SemaphoreType.DMA,
        ],
    )
    def k(table_hbm, idx_hbm, out_hbm, idx_v, rows0, rows1,
          sem0, sem1, osem0, osem1):
        wid = lax.axis_index("s") * 2 + lax.axis_index("c")
        base = wid * bpw
        pltpu.sync_copy(idx_hbm.at[pl.ds(off + base, bpw)], idx_v)
        rows = (rows0, rows1)
        sems = (sem0, sem1)
        osems = (osem0, osem1)

        def fire(step, buf):
            hnds = []
            for h in range(2):
                hnds.append(pltpu.async_copy(
                    table_hbm.at[idx_v.at[pl.ds(step * CB + h * CH, CH)]],
                    rows[buf].at[pl.ds(h * CH, CH)],
                    sems[buf]))
            return hnds

        pend_g = fire(0, 0)
        pend_o = [None, None]
        for c in range(nsteps):
            b = c & 1
            nb = (c + 1) & 1
            if c + 1 < nsteps:
                if pend_o[nb] is not None:
                    pend_o[nb].wait()
                    pend_o[nb] = None
                nxt = fire(c + 1, nb)
            for h in pend_g:
                h.wait()
            pend_o[b] = pltpu.async_copy(
                rows[b], out_hbm.at[pl.ds(base + c * CB, CB)], osems[b])
            if c + 1 < nsteps:
                pend_g = nxt
        for b in range(2):
            if pend_o[b] is not None:
                pend_o[b].wait()

    return k(table, idx)


def _shift1(x, carry):
    r = pltpu.roll(x, shift=1, axis=0)
    li = lax.broadcasted_iota(jnp.int32, x.shape, 0)
    return jnp.where(li == 0, carry, r)


def _make_phase_body(offt, nt, first, last):
    def body(hs_ref, accin_ref, chin_ref, wprev_ref, wcur_ref,
             w1212_ref, b1212_ref, *rest):
        if last:
            (w13_ref, w21_ref, w22_ref, w23_ref, bs3_ref, b13_ref,
             b21_ref, b22_ref, b23_ref, gwp_ref,
             accout_ref, chout_ref, of_ref, os_ref, ow_ref, acc12, ch) = rest
        else:
            accout_ref, chout_ref, acc12, ch = rest
        t = pl.program_id(0)

        @pl.when(t == 0)
        def _init():
            acc12[...] = accin_ref[...]
            ch[...] = chin_ref[...]

        hs = hs_ref[...]
        hs_prev = _shift1(hs, ch[...])
        ch[...] = hs[-1:, :]

        ppe = jnp.dot(hs_prev, wprev_ref[...],
                      preferred_element_type=jnp.float32)
        p_prev = ppe[:, :D]
        q_prev = ppe[:, D:2 * D]
        e_prev = ppe[:, 2 * D:]
        qe = jnp.dot(hs, wcur_ref[...], preferred_element_type=jnp.float32)
        q = qe[:, :D]
        e_cur = qe[:, D:]
        el = jnp.maximum(e_cur, 0.2 * e_cur)
        el_prev = jnp.maximum(e_prev, 0.2 * e_prev)

        a = 1.0 / (1.0 + jnp.exp(el - el_prev))
        msg = q + a * (q_prev - q)

        rin = jnp.concatenate([hs + p_prev, msg], axis=1)
        r12 = jnp.maximum(
            jnp.dot(rin, w1212_ref[...], preferred_element_type=jnp.float32)
            + b1212_ref[...], 0.0)

        edge = None
        if first and nt == 1:
            edge = (t == 0)
        elif first:
            edge = (t == 0)
        if last:
            e2 = (t == nt - 1)
            edge = e2 if edge is None else (edge | e2)

        if edge is None:
            acc12[...] += jnp.sum(r12, axis=0, keepdims=True)
        else:
            @pl.when(jnp.logical_not(edge))
            def _mid():
                acc12[...] += jnp.sum(r12, axis=0, keepdims=True)

            @pl.when(edge)
            def _edge():
                row = (offt + t) * T + lax.broadcasted_iota(
                    jnp.int32, (T, 1), 0)
                r1m = jnp.where(row < L, r12[:, :D], 0.0)
                r2m = jnp.where((row >= 1) & (row <= L - 1), r12[:, D:], 0.0)
                acc12[:, :D] += jnp.sum(r1m, axis=0, keepdims=True)
                acc12[:, D:] += jnp.sum(r2m, axis=0, keepdims=True)

        @pl.when(t == nt - 1)
        def _fin():
            accout_ref[...] = acc12[...]
            chout_ref[...] = ch[...]
            if last:
                m1 = acc12[:, :D] * (1.0 / L)
                m2 = acc12[:, D:] * (1.0 / (L - 1))
                rep1 = jnp.dot(m1, w21_ref[...],
                               preferred_element_type=jnp.float32) + b21_ref[...]
                rep2 = jnp.dot(m2, w22_ref[...],
                               preferred_element_type=jnp.float32) + b22_ref[...]
                h3 = jnp.maximum(
                    jnp.dot(bs3_ref[...], w13_ref[...],
                            preferred_element_type=jnp.float32)
                    + b13_ref[...], 0.0)
                rep3 = jnp.dot(h3, w23_ref[...],
                               preferred_element_type=jnp.float32) + b23_ref[...]
                gwp = gwp_ref[...]
                ew = jnp.exp(gwp - jnp.max(gwp))
                wv = ew / jnp.sum(ew)
                fused = (wv[:, 0:1] * rep1 + wv[:, 1:2] * rep2
                         + wv[:, 2:3] * rep3)
                of_ref[...] = fused
                os_ref[0:1, :] = rep1
                os_ref[1:2, :] = rep2
                os_ref[2:3, :] = rep3
                ow_ref[...] = wv

    return body


def _dense_phase(hs_k, acc, chv, proj, ro, offt, nt, first, last):
    full = lambda shape: pl.BlockSpec(shape, lambda t: (0, 0))
    v = full((1, D))
    in_specs = [
        pl.BlockSpec((T, D), lambda t: (t, 0)),
        full((1, 2 * D)),
        v,
        full((D, 3 * D)),
        full((D, 2 * D)),
        full((2 * D, 2 * D)),
        full((1, 2 * D)),
    ]
    args = (hs_k, acc, chv) + proj
    out_specs = [full((1, 2 * D)), v]
    out_shape = [jax.ShapeDtypeStruct((1, 2 * D), jnp.float32),
                 jax.ShapeDtypeStruct((1, D), jnp.float32)]
    if last:
        in_specs += [
            full((D, D)),
            full((D, D)), full((D, D)), full((D, D)),
            v,
            v,
            v, v, v,
            v,
        ]
        args += ro
        out_specs += [v, full((3, D)), v]
        out_shape += [jax.ShapeDtypeStruct((1, D), jnp.float32),
                      jax.ShapeDtypeStruct((3, D), jnp.float32),
                      jax.ShapeDtypeStruct((1, D), jnp.float32)]
    return pl.pallas_call(
        _make_phase_body(offt, nt, first, last),
        grid=(nt,),
        in_specs=in_specs,
        out_specs=out_specs,
        out_shape=out_shape,
        scratch_shapes=[pltpu.VMEM((1, 2 * D), jnp.float32),
                        pltpu.VMEM((1, D), jnp.float32)],
        compiler_params=pltpu.CompilerParams(
            dimension_semantics=("arbitrary",)),
    )(*args)


def kernel(items, params):
    idxp = jnp.pad(items, (0, LP - L))

    gidx = jnp.arange(D, dtype=jnp.int32) // DH
    bones = (gidx[:, None] == gidx[None, :]).astype(jnp.float32)
    wel = jnp.dot(params['W_up1'] * params['al_up1'].reshape(1, D), bones)
    wprev = jnp.concatenate([params['W_seq1'], params['W_up1'], wel], axis=1)
    wcur = jnp.concatenate([params['W_up1'], wel], axis=1)
    z = jnp.zeros((D, D), jnp.float32)
    w1212 = jnp.block([[params['ro1_W1'], z], [z, params['ro2_W1']]])
    bsum1 = params['b_seq1'] + params['b_down1']
    bsum2 = params['b_seq2'] + params['b_down2'] + params['b_up1']
    b1212 = jnp.concatenate(
        [jnp.dot(bsum1, params['ro1_W1']) + params['ro1_b1'],
         jnp.dot(bsum2, params['ro2_W1']) + params['ro2_b1']]).reshape(1, 2 * D)
    r = lambda x: x.reshape(1, D)
    proj = (wprev, wcur, w1212, b1212)
    ro = (params['ro3_W1'],
          params['ro1_W2'], params['ro2_W2'], params['ro3_W2'],
          r(params['b_seq3'] + params['b_up2']),
          r(params['ro3_b1']),
          r(params['ro1_b2']), r(params['ro2_b2']), r(params['ro3_b2']),
          jnp.concatenate([params['gw'],
                           jnp.full((D - 3,), -jnp.inf, jnp.float32)]
                          ).reshape(1, D))

    segs = []
    offt = 0
    for nt in PHASES:
        segs.append((offt, nt))
        offt += nt
    hs_parts = [_sc_gather(params['emb'],
                           idxp if (o + n) * T > L else items, o, n)
                for o, n in segs]

    acc = jnp.zeros((1, 2 * D), jnp.float32)
    chv = jnp.zeros((1, D), jnp.float32)
    for k_i, (o, n) in enumerate(segs):
        res = _dense_phase(hs_parts[k_i], acc, chv, proj, ro, o, n,
                           first=(k_i == 0), last=(k_i == len(segs) - 1))
        acc, chv = res[0], res[1]
    of, os_, ow = res[2], res[3], res[4]
    return of.reshape(D), os_, ow[0, :3]


# speedup vs baseline: 587.2226x; 1.0063x over previous
import functools

import jax
import jax.numpy as jnp
from jax import lax
from jax.experimental import pallas as pl
from jax.experimental.pallas import tpu as pltpu
from jax.experimental.pallas import tpu_sc as plsc

L = 50000
D = 128
DH = 16
NW = 32
LP = 50176
CH = 112
CB = 2 * CH
T = 1024
G = LP // T
PHASES = (7, 14, 14, 14)


def _sc_gather(table, idx, offt, nt):
    mesh = plsc.VectorSubcoreMesh(core_axis_name="c", subcore_axis_name="s")
    bpw = nt * T // NW
    nsteps = bpw // CB
    off = offt * T

    @functools.partial(
        pl.kernel,
        out_type=jax.ShapeDtypeStruct((nt * T, D), jnp.float32),
        mesh=mesh,
        scratch_types=[
            pltpu.VMEM((bpw,), jnp.int32),
            pltpu.VMEM((CB, D), jnp.float32),
            pltpu.VMEM((CB, D), jnp.float32),
            pltpu.SemaphoreType.DMA,
            pltpu.SemaphoreType.DMA,
            pltpu.SemaphoreType.DMA,
            pltpu.SemaphoreType.DMA,
        ],
    )
    def k(table_hbm, idx_hbm, out_hbm, idx_v, rows0, rows1,
          sem0, sem1, osem0, osem1):
        wid = lax.axis_index("s") * 2 + lax.axis_index("c")
        base = wid * bpw
        pltpu.sync_copy(idx_hbm.at[pl.ds(off + base, bpw)], idx_v)
        rows = (rows0, rows1)
        sems = (sem0, sem1)
        osems = (osem0, osem1)

        def fire(step, buf):
            hnds = []
            for h in range(2):
                hnds.append(pltpu.async_copy(
                    table_hbm.at[idx_v.at[pl.ds(step * CB + h * CH, CH)]],
                    rows[buf].at[pl.ds(h * CH, CH)],
                    sems[buf]))
            return hnds

        pend_g = fire(0, 0)
        pend_o = [None, None]
        for c in range(nsteps):
            b = c & 1
            nb = (c + 1) & 1
            if c + 1 < nsteps:
                if pend_o[nb] is not None:
                    pend_o[nb].wait()
                    pend_o[nb] = None
                nxt = fire(c + 1, nb)
            for h in pend_g:
                h.wait()
            pend_o[b] = pltpu.async_copy(
                rows[b], out_hbm.at[pl.ds(base + c * CB, CB)], osems[b])
            if c + 1 < nsteps:
                pend_g = nxt
        for b in range(2):
            if pend_o[b] is not None:
                pend_o[b].wait()

    return k(table, idx)


def _shift1(x, carry):
    r = pltpu.roll(x, shift=1, axis=0)
    li = lax.broadcasted_iota(jnp.int32, x.shape, 0)
    return jnp.where(li == 0, carry, r)


def _make_phase_body(offt, nt, first, last):
    def body(hs_ref, accin_ref, chin_ref, wprev_ref, wcur_ref,
             w1212_ref, b1212_ref, *rest):
        if last:
            (w13_ref, w21_ref, w22_ref, w23_ref, bs3_ref, b13_ref,
             b21_ref, b22_ref, b23_ref, gwp_ref,
             accout_ref, chout_ref, of_ref, os_ref, ow_ref, acc12, ch) = rest
        else:
            accout_ref, chout_ref, acc12, ch = rest
        t = pl.program_id(0)

        @pl.when(t == 0)
        def _init():
            acc12[...] = accin_ref[...]
            ch[...] = chin_ref[...].astype(jnp.bfloat16)

        hs = hs_ref[...]
        hs_b = hs.astype(jnp.bfloat16)
        hs_prev_b = _shift1(hs_b, ch[...])
        ch[...] = hs_b[-1:, :]

        ppe = jnp.dot(hs_prev_b, wprev_ref[...],
                      preferred_element_type=jnp.float32)
        p_prev = ppe[:, :D]
        q_prev = ppe[:, D:2 * D]
        e_prev = ppe[:, 2 * D:]
        qe = jnp.dot(hs_b, wcur_ref[...], preferred_element_type=jnp.float32)
        q = qe[:, :D]
        e_cur = qe[:, D:]
        el = jnp.maximum(e_cur, 0.2 * e_cur)
        el_prev = jnp.maximum(e_prev, 0.2 * e_prev)

        a = 1.0 / (1.0 + jnp.exp(el - el_prev))
        msg = q + a * (q_prev - q)

        rin = jnp.concatenate([hs + p_prev, msg], axis=1)
        r12 = jnp.maximum(
            jnp.dot(rin, w1212_ref[...], preferred_element_type=jnp.float32)
            + b1212_ref[...], 0.0)

        edge = None
        if first and nt == 1:
            edge = (t == 0)
        elif first:
            edge = (t == 0)
        if last:
            e2 = (t == nt - 1)
            edge = e2 if edge is None else (edge | e2)

        if edge is None:
            acc12[...] += jnp.sum(r12, axis=0, keepdims=True)
        else:
            @pl.when(jnp.logical_not(edge))
            def _mid():
                acc12[...] += jnp.sum(r12, axis=0, keepdims=True)

            @pl.when(edge)
            def _edge():
                row = (offt + t) * T + lax.broadcasted_iota(
                    jnp.int32, (T, 1), 0)
                r1m = jnp.where(row < L, r12[:, :D], 0.0)
                r2m = jnp.where((row >= 1) & (row <= L - 1), r12[:, D:], 0.0)
                acc12[:, :D] += jnp.sum(r1m, axis=0, keepdims=True)
                acc12[:, D:] += jnp.sum(r2m, axis=0, keepdims=True)

        @pl.when(t == nt - 1)
        def _fin():
            accout_ref[...] = acc12[...]
            chout_ref[...] = ch[...].astype(jnp.float32)
            if last:
                m1 = acc12[:, :D] * (1.0 / L)
                m2 = acc12[:, D:] * (1.0 / (L - 1))
                rep1 = jnp.dot(m1, w21_ref[...],
                               preferred_element_type=jnp.float32) + b21_ref[...]
                rep2 = jnp.dot(m2, w22_ref[...],
                               preferred_element_type=jnp.float32) + b22_ref[...]
                h3 = jnp.maximum(
                    jnp.dot(bs3_ref[...], w13_ref[...],
                            preferred_element_type=jnp.float32)
                    + b13_ref[...], 0.0)
                rep3 = jnp.dot(h3, w23_ref[...],
                               preferred_element_type=jnp.float32) + b23_ref[...]
                gwp = gwp_ref[...]
                ew = jnp.exp(gwp - jnp.max(gwp))
                wv = ew / jnp.sum(ew)
                fused = (wv[:, 0:1] * rep1 + wv[:, 1:2] * rep2
                         + wv[:, 2:3] * rep3)
                of_ref[...] = fused
                os_ref[0:1, :] = rep1
                os_ref[1:2, :] = rep2
                os_ref[2:3, :] = rep3
                ow_ref[...] = wv

    return body


def _dense_phase(hs_k, acc, chv, proj, ro, offt, nt, first, last):
    full = lambda shape: pl.BlockSpec(shape, lambda t: (0, 0))
    v = full((1, D))
    in_specs = [
        pl.BlockSpec((T, D), lambda t: (t, 0)),
        full((1, 2 * D)),
        v,
        full((D, 3 * D)),
        full((D, 2 * D)),
        full((2 * D, 2 * D)),
        full((1, 2 * D)),
    ]
    args = (hs_k, acc, chv) + proj
    out_specs = [full((1, 2 * D)), v]
    out_shape = [jax.ShapeDtypeStruct((1, 2 * D), jnp.float32),
                 jax.ShapeDtypeStruct((1, D), jnp.float32)]
    if last:
        in_specs += [
            full((D, D)),
            full((D, D)), full((D, D)), full((D, D)),
            v,
            v,
            v, v, v,
            v,
        ]
        args += ro
        out_specs += [v, full((3, D)), v]
        out_shape += [jax.ShapeDtypeStruct((1, D), jnp.float32),
                      jax.ShapeDtypeStruct((3, D), jnp.float32),
                      jax.ShapeDtypeStruct((1, D), jnp.float32)]
    return pl.pallas_call(
        _make_phase_body(offt, nt, first, last),
        grid=(nt,),
        in_specs=in_specs,
        out_specs=out_specs,
        out_shape=out_shape,
        scratch_shapes=[pltpu.VMEM((1, 2 * D), jnp.float32),
                        pltpu.VMEM((1, D), jnp.bfloat16)],
        compiler_params=pltpu.CompilerParams(
            dimension_semantics=("arbitrary",)),
    )(*args)


def kernel(items, params):
    idxp = jnp.pad(items, (0, LP - L))

    gidx = jnp.arange(D, dtype=jnp.int32) // DH
    bones = (gidx[:, None] == gidx[None, :]).astype(jnp.float32)
    wel = jnp.dot(params['W_up1'] * params['al_up1'].reshape(1, D), bones)
    wprev = jnp.concatenate([params['W_seq1'], params['W_up1'], wel],
                            axis=1).astype(jnp.bfloat16)
    wcur = jnp.concatenate([params['W_up1'], wel],
                           axis=1).astype(jnp.bfloat16)
    z = jnp.zeros((D, D), jnp.float32)
    w1212 = jnp.block([[params['ro1_W1'], z], [z, params['ro2_W1']]])
    bsum1 = params['b_seq1'] + params['b_down1']
    bsum2 = params['b_seq2'] + params['b_down2'] + params['b_up1']
    b1212 = jnp.concatenate(
        [jnp.dot(bsum1, params['ro1_W1']) + params['ro1_b1'],
         jnp.dot(bsum2, params['ro2_W1']) + params['ro2_b1']]).reshape(1, 2 * D)
    r = lambda x: x.reshape(1, D)
    proj = (wprev, wcur, w1212, b1212)
    ro = (params['ro3_W1'],
          params['ro1_W2'], params['ro2_W2'], params['ro3_W2'],
          r(params['b_seq3'] + params['b_up2']),
          r(params['ro3_b1']),
          r(params['ro1_b2']), r(params['ro2_b2']), r(params['ro3_b2']),
          jnp.concatenate([params['gw'],
                           jnp.full((D - 3,), -jnp.inf, jnp.float32)]
                          ).reshape(1, D))

    segs = []
    offt = 0
    for nt in PHASES:
        segs.append((offt, nt))
        offt += nt
    hs_parts = [_sc_gather(params['emb'],
                           idxp if (o + n) * T > L else items, o, n)
                for o, n in segs]

    acc = jnp.zeros((1, 2 * D), jnp.float32)
    chv = jnp.zeros((1, D), jnp.float32)
    for k_i, (o, n) in enumerate(segs):
        res = _dense_phase(hs_parts[k_i], acc, chv, proj, ro, o, n,
                           first=(k_i == 0), last=(k_i == len(segs) - 1))
        acc, chv = res[0], res[1]
    of, os_, ow = res[2], res[3], res[4]
    return of.reshape(D), os_, ow[0, :3]


# speedup vs baseline: 590.2211x; 1.0051x over previous
import functools

import jax
import jax.numpy as jnp
from jax import lax
from jax.experimental import pallas as pl
from jax.experimental.pallas import tpu as pltpu
from jax.experimental.pallas import tpu_sc as plsc

L = 50000
D = 128
DH = 16
NW = 32
LP = 50176
CH = 112
CB = 2 * CH
T = 1024
G = LP // T
PHASES = (7, 14, 14, 14)


def _sc_gather(table, idx, offt, nt):
    mesh = plsc.VectorSubcoreMesh(core_axis_name="c", subcore_axis_name="s")
    bpw = nt * T // NW
    nsteps = bpw // CB
    off = offt * T

    @functools.partial(
        pl.kernel,
        out_type=jax.ShapeDtypeStruct((nt * T, D), jnp.float32),
        mesh=mesh,
        scratch_types=[
            pltpu.VMEM((bpw,), jnp.int32),
            pltpu.VMEM((CB, D), jnp.float32),
            pltpu.VMEM((CB, D), jnp.float32),
            pltpu.SemaphoreType.DMA,
            pltpu.SemaphoreType.DMA,
            pltpu.SemaphoreType.DMA,
            pltpu.SemaphoreType.DMA,
        ],
    )
    def k(table_hbm, idx_hbm, out_hbm, idx_v, rows0, rows1,
          sem0, sem1, osem0, osem1):
        wid = lax.axis_index("s") * 2 + lax.axis_index("c")
        base = wid * bpw
        pltpu.sync_copy(idx_hbm.at[pl.ds(off + base, bpw)], idx_v)
        rows = (rows0, rows1)
        sems = (sem0, sem1)
        osems = (osem0, osem1)

        def fire(step, buf):
            hnds = []
            for h in range(2):
                hnds.append(pltpu.async_copy(
                    table_hbm.at[idx_v.at[pl.ds(step * CB + h * CH, CH)]],
                    rows[buf].at[pl.ds(h * CH, CH)],
                    sems[buf]))
            return hnds

        pend_g = fire(0, 0)
        pend_o = [None, None]
        for c in range(nsteps):
            b = c & 1
            nb = (c + 1) & 1
            if c + 1 < nsteps:
                if pend_o[nb] is not None:
                    pend_o[nb].wait()
                    pend_o[nb] = None
                nxt = fire(c + 1, nb)
            for h in pend_g:
                h.wait()
            pend_o[b] = pltpu.async_copy(
                rows[b], out_hbm.at[pl.ds(base + c * CB, CB)], osems[b])
            if c + 1 < nsteps:
                pend_g = nxt
        for b in range(2):
            if pend_o[b] is not None:
                pend_o[b].wait()

    return k(table, idx)


def _shift1(x, carry):
    r = pltpu.roll(x, shift=1, axis=0)
    li = lax.broadcasted_iota(jnp.int32, x.shape, 0)
    return jnp.where(li == 0, carry, r)


def _make_phase_body(offt, nt, first, last):
    def body(hs_ref, accin_ref, chin_ref, wprev_ref, wcur_ref,
             w1212_ref, b1212_ref, *rest):
        if last:
            (w13_ref, w21_ref, w22_ref, w23_ref, bs3_ref, b13_ref,
             b21_ref, b22_ref, b23_ref, gwp_ref,
             accout_ref, chout_ref, of_ref, os_ref, ow_ref, acc12, ch) = rest
        else:
            accout_ref, chout_ref, acc12, ch = rest
        t = pl.program_id(0)

        @pl.when(t == 0)
        def _init():
            acc12[...] = accin_ref[...]
            ch[...] = chin_ref[...].astype(jnp.bfloat16)

        hs = hs_ref[...]
        hs_b = hs.astype(jnp.bfloat16)
        hs_prev_b = _shift1(hs_b, ch[...])
        ch[...] = hs_b[-1:, :]

        ppe = jnp.dot(hs_prev_b, wprev_ref[...],
                      preferred_element_type=jnp.float32)
        p_prev = ppe[:, :D]
        q_prev = ppe[:, D:2 * D]
        e_prev = ppe[:, 2 * D:]
        qe = jnp.dot(hs_b, wcur_ref[...], preferred_element_type=jnp.float32)
        q = qe[:, :D]
        e_cur = qe[:, D:]
        el = jnp.maximum(e_cur, 0.2 * e_cur)
        el_prev = jnp.maximum(e_prev, 0.2 * e_prev)

        a = 1.0 / (1.0 + jnp.exp(el - el_prev))
        msg = q + a * (q_prev - q)

        rin = jnp.concatenate([hs + p_prev, msg],
                              axis=1).astype(jnp.bfloat16)
        r12 = jnp.maximum(
            jnp.dot(rin, w1212_ref[...], preferred_element_type=jnp.float32)
            + b1212_ref[...], 0.0)

        edge = None
        if first and nt == 1:
            edge = (t == 0)
        elif first:
            edge = (t == 0)
        if last:
            e2 = (t == nt - 1)
            edge = e2 if edge is None else (edge | e2)

        if edge is None:
            acc12[...] += jnp.sum(r12, axis=0, keepdims=True)
        else:
            @pl.when(jnp.logical_not(edge))
            def _mid():
                acc12[...] += jnp.sum(r12, axis=0, keepdims=True)

            @pl.when(edge)
            def _edge():
                row = (offt + t) * T + lax.broadcasted_iota(
                    jnp.int32, (T, 1), 0)
                r1m = jnp.where(row < L, r12[:, :D], 0.0)
                r2m = jnp.where((row >= 1) & (row <= L - 1), r12[:, D:], 0.0)
                acc12[:, :D] += jnp.sum(r1m, axis=0, keepdims=True)
                acc12[:, D:] += jnp.sum(r2m, axis=0, keepdims=True)

        @pl.when(t == nt - 1)
        def _fin():
            accout_ref[...] = acc12[...]
            chout_ref[...] = ch[...].astype(jnp.float32)
            if last:
                m1 = acc12[:, :D] * (1.0 / L)
                m2 = acc12[:, D:] * (1.0 / (L - 1))
                rep1 = jnp.dot(m1, w21_ref[...],
                               preferred_element_type=jnp.float32) + b21_ref[...]
                rep2 = jnp.dot(m2, w22_ref[...],
                               preferred_element_type=jnp.float32) + b22_ref[...]
                h3 = jnp.maximum(
                    jnp.dot(bs3_ref[...], w13_ref[...],
                            preferred_element_type=jnp.float32)
                    + b13_ref[...], 0.0)
                rep3 = jnp.dot(h3, w23_ref[...],
                               preferred_element_type=jnp.float32) + b23_ref[...]
                gwp = gwp_ref[...]
                ew = jnp.exp(gwp - jnp.max(gwp))
                wv = ew / jnp.sum(ew)
                fused = (wv[:, 0:1] * rep1 + wv[:, 1:2] * rep2
                         + wv[:, 2:3] * rep3)
                of_ref[...] = fused
                os_ref[0:1, :] = rep1
                os_ref[1:2, :] = rep2
                os_ref[2:3, :] = rep3
                ow_ref[...] = wv

    return body


def _dense_phase(hs_k, acc, chv, proj, ro, offt, nt, first, last):
    full = lambda shape: pl.BlockSpec(shape, lambda t: (0, 0))
    v = full((1, D))
    in_specs = [
        pl.BlockSpec((T, D), lambda t: (t, 0)),
        full((1, 2 * D)),
        v,
        full((D, 3 * D)),
        full((D, 2 * D)),
        full((2 * D, 2 * D)),
        full((1, 2 * D)),
    ]
    args = (hs_k, acc, chv) + proj
    out_specs = [full((1, 2 * D)), v]
    out_shape = [jax.ShapeDtypeStruct((1, 2 * D), jnp.float32),
                 jax.ShapeDtypeStruct((1, D), jnp.float32)]
    if last:
        in_specs += [
            full((D, D)),
            full((D, D)), full((D, D)), full((D, D)),
            v,
            v,
            v, v, v,
            v,
        ]
        args += ro
        out_specs += [v, full((3, D)), v]
        out_shape += [jax.ShapeDtypeStruct((1, D), jnp.float32),
                      jax.ShapeDtypeStruct((3, D), jnp.float32),
                      jax.ShapeDtypeStruct((1, D), jnp.float32)]
    return pl.pallas_call(
        _make_phase_body(offt, nt, first, last),
        grid=(nt,),
        in_specs=in_specs,
        out_specs=out_specs,
        out_shape=out_shape,
        scratch_shapes=[pltpu.VMEM((1, 2 * D), jnp.float32),
                        pltpu.VMEM((1, D), jnp.bfloat16)],
        compiler_params=pltpu.CompilerParams(
            dimension_semantics=("arbitrary",)),
    )(*args)


def kernel(items, params):
    idxp = jnp.pad(items, (0, LP - L))

    gidx = jnp.arange(D, dtype=jnp.int32) // DH
    bones = (gidx[:, None] == gidx[None, :]).astype(jnp.float32)
    wel = jnp.dot(params['W_up1'] * params['al_up1'].reshape(1, D), bones)
    wprev = jnp.concatenate([params['W_seq1'], params['W_up1'], wel],
                            axis=1).astype(jnp.bfloat16)
    wcur = jnp.concatenate([params['W_up1'], wel],
                           axis=1).astype(jnp.bfloat16)
    z = jnp.zeros((D, D), jnp.float32)
    w1212 = jnp.block([[params['ro1_W1'], z],
                       [z, params['ro2_W1']]]).astype(jnp.bfloat16)
    bsum1 = params['b_seq1'] + params['b_down1']
    bsum2 = params['b_seq2'] + params['b_down2'] + params['b_up1']
    b1212 = jnp.concatenate(
        [jnp.dot(bsum1, params['ro1_W1']) + params['ro1_b1'],
         jnp.dot(bsum2, params['ro2_W1']) + params['ro2_b1']]).reshape(1, 2 * D)
    r = lambda x: x.reshape(1, D)
    proj = (wprev, wcur, w1212, b1212)
    ro = (params['ro3_W1'],
          params['ro1_W2'], params['ro2_W2'], params['ro3_W2'],
          r(params['b_seq3'] + params['b_up2']),
          r(params['ro3_b1']),
          r(params['ro1_b2']), r(params['ro2_b2']), r(params['ro3_b2']),
          jnp.concatenate([params['gw'],
                           jnp.full((D - 3,), -jnp.inf, jnp.float32)]
                          ).reshape(1, D))

    segs = []
    offt = 0
    for nt in PHASES:
        segs.append((offt, nt))
        offt += nt
    hs_parts = [_sc_gather(params['emb'],
                           idxp if (o + n) * T > L else items, o, n)
                for o, n in segs]

    acc = jnp.zeros((1, 2 * D), jnp.float32)
    chv = jnp.zeros((1, D), jnp.float32)
    for k_i, (o, n) in enumerate(segs):
        res = _dense_phase(hs_parts[k_i], acc, chv, proj, ro, o, n,
                           first=(k_i == 0), last=(k_i == len(segs) - 1))
        acc, chv = res[0], res[1]
    of, os_, ow = res[2], res[3], res[4]
    return of.reshape(D), os_, ow[0, :3]


# speedup vs baseline: 665.7030x; 1.1279x over previous
import functools

import jax
import jax.numpy as jnp
from jax import lax
from jax.experimental import pallas as pl
from jax.experimental.pallas import tpu as pltpu
from jax.experimental.pallas import tpu_sc as plsc

L = 50000
D = 128
DH = 16
NW = 32
LP = 50176
T = 1792
G = LP // T
PHASES = (4, 8, 8, 8)


def _pick_cb(bpw):
    for d in range(min(bpw, 256), 0, -16):
        if bpw % d == 0 and (d // 2) % 8 == 0 and d // 2 <= 128:
            return d
    raise ValueError(bpw)


def _sc_gather(table, idx, offt, nt):
    mesh = plsc.VectorSubcoreMesh(core_axis_name="c", subcore_axis_name="s")
    bpw = nt * T // NW
    cb = _pick_cb(bpw)
    ch_rows = cb // 2
    nsteps = bpw // cb
    off = offt * T

    @functools.partial(
        pl.kernel,
        out_type=jax.ShapeDtypeStruct((nt * T, D), jnp.float32),
        mesh=mesh,
        scratch_types=[
            pltpu.VMEM((bpw,), jnp.int32),
            pltpu.VMEM((cb, D), jnp.float32),
            pltpu.VMEM((cb, D), jnp.float32),
            pltpu.SemaphoreType.DMA,
            pltpu.SemaphoreType.DMA,
            pltpu.SemaphoreType.DMA,
            pltpu.SemaphoreType.DMA,
        ],
    )
    def k(table_hbm, idx_hbm, out_hbm, idx_v, rows0, rows1,
          sem0, sem1, osem0, osem1):
        wid = lax.axis_index("s") * 2 + lax.axis_index("c")
        base = wid * bpw
        pltpu.sync_copy(idx_hbm.at[pl.ds(off + base, bpw)], idx_v)
        rows = (rows0, rows1)
        sems = (sem0, sem1)
        osems = (osem0, osem1)

        def fire(step, buf):
            hnds = []
            for h in range(2):
                hnds.append(pltpu.async_copy(
                    table_hbm.at[idx_v.at[pl.ds(step * cb + h * ch_rows,
                                                ch_rows)]],
                    rows[buf].at[pl.ds(h * ch_rows, ch_rows)],
                    sems[buf]))
            return hnds

        pend_g = fire(0, 0)
        pend_o = [None, None]
        for c in range(nsteps):
            b = c & 1
            nb = (c + 1) & 1
            if c + 1 < nsteps:
                if pend_o[nb] is not None:
                    pend_o[nb].wait()
                    pend_o[nb] = None
                nxt = fire(c + 1, nb)
            for h in pend_g:
                h.wait()
            pend_o[b] = pltpu.async_copy(
                rows[b], out_hbm.at[pl.ds(base + c * cb, cb)], osems[b])
            if c + 1 < nsteps:
                pend_g = nxt
        for b in range(2):
            if pend_o[b] is not None:
                pend_o[b].wait()

    return k(table, idx)


def _shift1(x, carry):
    r = pltpu.roll(x, shift=1, axis=0)
    li = lax.broadcasted_iota(jnp.int32, x.shape, 0)
    return jnp.where(li == 0, carry, r)


def _make_phase_body(offt, nt, first, last):
    def body(hs_ref, accin_ref, chin_ref, wprev_ref, wcur_ref,
             w1212_ref, b1212_ref, *rest):
        if last:
            (w13_ref, w21_ref, w22_ref, w23_ref, bs3_ref, b13_ref,
             b21_ref, b22_ref, b23_ref, gwp_ref,
             accout_ref, chout_ref, of_ref, os_ref, ow_ref, acc12, ch) = rest
        else:
            accout_ref, chout_ref, acc12, ch = rest
        t = pl.program_id(0)

        @pl.when(t == 0)
        def _init():
            acc12[...] = accin_ref[...]
            ch[...] = chin_ref[...].astype(jnp.bfloat16)

        hs = hs_ref[...]
        hs_b = hs.astype(jnp.bfloat16)
        hs_prev_b = _shift1(hs_b, ch[...])
        ch[...] = hs_b[-1:, :]

        ppe = jnp.dot(hs_prev_b, wprev_ref[...],
                      preferred_element_type=jnp.float32)
        p_prev = ppe[:, :D]
        q_prev = ppe[:, D:2 * D]
        e_prev = ppe[:, 2 * D:]
        qe = jnp.dot(hs_b, wcur_ref[...], preferred_element_type=jnp.float32)
        q = qe[:, :D]
        e_cur = qe[:, D:]
        el = jnp.maximum(e_cur, 0.2 * e_cur)
        el_prev = jnp.maximum(e_prev, 0.2 * e_prev)

        a = 1.0 / (1.0 + jnp.exp(el - el_prev))
        msg = q + a * (q_prev - q)

        rin = jnp.concatenate([hs + p_prev, msg],
                              axis=1).astype(jnp.bfloat16)
        r12 = jnp.maximum(
            jnp.dot(rin, w1212_ref[...], preferred_element_type=jnp.float32)
            + b1212_ref[...], 0.0)

        edge = None
        if first and nt == 1:
            edge = (t == 0)
        elif first:
            edge = (t == 0)
        if last:
            e2 = (t == nt - 1)
            edge = e2 if edge is None else (edge | e2)

        if edge is None:
            acc12[...] += jnp.sum(r12, axis=0, keepdims=True)
        else:
            @pl.when(jnp.logical_not(edge))
            def _mid():
                acc12[...] += jnp.sum(r12, axis=0, keepdims=True)

            @pl.when(edge)
            def _edge():
                row = (offt + t) * T + lax.broadcasted_iota(
                    jnp.int32, (T, 1), 0)
                r1m = jnp.where(row < L, r12[:, :D], 0.0)
                r2m = jnp.where((row >= 1) & (row <= L - 1), r12[:, D:], 0.0)
                acc12[:, :D] += jnp.sum(r1m, axis=0, keepdims=True)
                acc12[:, D:] += jnp.sum(r2m, axis=0, keepdims=True)

        @pl.when(t == nt - 1)
        def _fin():
            accout_ref[...] = acc12[...]
            chout_ref[...] = ch[...].astype(jnp.float32)
            if last:
                m1 = acc12[:, :D] * (1.0 / L)
                m2 = acc12[:, D:] * (1.0 / (L - 1))
                rep1 = jnp.dot(m1, w21_ref[...],
                               preferred_element_type=jnp.float32) + b21_ref[...]
                rep2 = jnp.dot(m2, w22_ref[...],
                               preferred_element_type=jnp.float32) + b22_ref[...]
                h3 = jnp.maximum(
                    jnp.dot(bs3_ref[...], w13_ref[...],
                            preferred_element_type=jnp.float32)
                    + b13_ref[...], 0.0)
                rep3 = jnp.dot(h3, w23_ref[...],
                               preferred_element_type=jnp.float32) + b23_ref[...]
                gwp = gwp_ref[...]
                ew = jnp.exp(gwp - jnp.max(gwp))
                wv = ew / jnp.sum(ew)
                fused = (wv[:, 0:1] * rep1 + wv[:, 1:2] * rep2
                         + wv[:, 2:3] * rep3)
                of_ref[...] = fused
                os_ref[0:1, :] = rep1
                os_ref[1:2, :] = rep2
                os_ref[2:3, :] = rep3
                ow_ref[...] = wv

    return body


def _dense_phase(hs_k, acc, chv, proj, ro, offt, nt, first, last):
    full = lambda shape: pl.BlockSpec(shape, lambda t: (0, 0))
    v = full((1, D))
    in_specs = [
        pl.BlockSpec((T, D), lambda t: (t, 0)),
        full((1, 2 * D)),
        v,
        full((D, 3 * D)),
        full((D, 2 * D)),
        full((2 * D, 2 * D)),
        full((1, 2 * D)),
    ]
    args = (hs_k, acc, chv) + proj
    out_specs = [full((1, 2 * D)), v]
    out_shape = [jax.ShapeDtypeStruct((1, 2 * D), jnp.float32),
                 jax.ShapeDtypeStruct((1, D), jnp.float32)]
    if last:
        in_specs += [
            full((D, D)),
            full((D, D)), full((D, D)), full((D, D)),
            v,
            v,
            v, v, v,
            v,
        ]
        args += ro
        out_specs += [v, full((3, D)), v]
        out_shape += [jax.ShapeDtypeStruct((1, D), jnp.float32),
                      jax.ShapeDtypeStruct((3, D), jnp.float32),
                      jax.ShapeDtypeStruct((1, D), jnp.float32)]
    return pl.pallas_call(
        _make_phase_body(offt, nt, first, last),
        grid=(nt,),
        in_specs=in_specs,
        out_specs=out_specs,
        out_shape=out_shape,
        scratch_shapes=[pltpu.VMEM((1, 2 * D), jnp.float32),
                        pltpu.VMEM((1, D), jnp.bfloat16)],
        compiler_params=pltpu.CompilerParams(
            dimension_semantics=("arbitrary",)),
    )(*args)


def kernel(items, params):
    idxp = jnp.pad(items, (0, LP - L))

    gidx = jnp.arange(D, dtype=jnp.int32) // DH
    bones = (gidx[:, None] == gidx[None, :]).astype(jnp.float32)
    wel = jnp.dot(params['W_up1'] * params['al_up1'].reshape(1, D), bones)
    wprev = jnp.concatenate([params['W_seq1'], params['W_up1'], wel],
                            axis=1).astype(jnp.bfloat16)
    wcur = jnp.concatenate([params['W_up1'], wel],
                           axis=1).astype(jnp.bfloat16)
    z = jnp.zeros((D, D), jnp.float32)
    w1212 = jnp.block([[params['ro1_W1'], z],
                       [z, params['ro2_W1']]]).astype(jnp.bfloat16)
    bsum1 = params['b_seq1'] + params['b_down1']
    bsum2 = params['b_seq2'] + params['b_down2'] + params['b_up1']
    b1212 = jnp.concatenate(
        [jnp.dot(bsum1, params['ro1_W1']) + params['ro1_b1'],
         jnp.dot(bsum2, params['ro2_W1']) + params['ro2_b1']]).reshape(1, 2 * D)
    r = lambda x: x.reshape(1, D)
    proj = (wprev, wcur, w1212, b1212)
    ro = (params['ro3_W1'],
          params['ro1_W2'], params['ro2_W2'], params['ro3_W2'],
          r(params['b_seq3'] + params['b_up2']),
          r(params['ro3_b1']),
          r(params['ro1_b2']), r(params['ro2_b2']), r(params['ro3_b2']),
          jnp.concatenate([params['gw'],
                           jnp.full((D - 3,), -jnp.inf, jnp.float32)]
                          ).reshape(1, D))

    segs = []
    offt = 0
    for nt in PHASES:
        segs.append((offt, nt))
        offt += nt
    hs_parts = [_sc_gather(params['emb'],
                           idxp if (o + n) * T > L else items, o, n)
                for o, n in segs]

    acc = jnp.zeros((1, 2 * D), jnp.float32)
    chv = jnp.zeros((1, D), jnp.float32)
    for k_i, (o, n) in enumerate(segs):
        res = _dense_phase(hs_parts[k_i], acc, chv, proj, ro, o, n,
                           first=(k_i == 0), last=(k_i == len(segs) - 1))
        acc, chv = res[0], res[1]
    of, os_, ow = res[2], res[3], res[4]
    return of.reshape(D), os_, ow[0, :3]


# speedup vs baseline: 690.2096x; 1.0368x over previous
import functools

import jax
import jax.numpy as jnp
from jax import lax
from jax.experimental import pallas as pl
from jax.experimental.pallas import tpu as pltpu
from jax.experimental.pallas import tpu_sc as plsc

L = 50000
D = 128
DH = 16
NW = 32
LP = 50176
T = 3584
G = LP // T
PHASES = (2, 4, 4, 4)


def _pick_cb(bpw):
    for d in range(min(bpw, 256), 0, -16):
        if bpw % d == 0 and (d // 2) % 8 == 0 and d // 2 <= 128:
            return d
    raise ValueError(bpw)


def _sc_gather(table, idx, offt, nt):
    mesh = plsc.VectorSubcoreMesh(core_axis_name="c", subcore_axis_name="s")
    bpw = nt * T // NW
    cb = _pick_cb(bpw)
    ch_rows = cb // 2
    nsteps = bpw // cb
    off = offt * T

    @functools.partial(
        pl.kernel,
        out_type=jax.ShapeDtypeStruct((nt * T, D), jnp.float32),
        mesh=mesh,
        scratch_types=[
            pltpu.VMEM((bpw,), jnp.int32),
            pltpu.VMEM((cb, D), jnp.float32),
            pltpu.VMEM((cb, D), jnp.float32),
            pltpu.SemaphoreType.DMA,
            pltpu.SemaphoreType.DMA,
            pltpu.SemaphoreType.DMA,
            pltpu.SemaphoreType.DMA,
        ],
    )
    def k(table_hbm, idx_hbm, out_hbm, idx_v, rows0, rows1,
          sem0, sem1, osem0, osem1):
        wid = lax.axis_index("s") * 2 + lax.axis_index("c")
        base = wid * bpw
        pltpu.sync_copy(idx_hbm.at[pl.ds(off + base, bpw)], idx_v)
        rows = (rows0, rows1)
        sems = (sem0, sem1)
        osems = (osem0, osem1)

        def fire(step, buf):
            hnds = []
            for h in range(2):
                hnds.append(pltpu.async_copy(
                    table_hbm.at[idx_v.at[pl.ds(step * cb + h * ch_rows,
                                                ch_rows)]],
                    rows[buf].at[pl.ds(h * ch_rows, ch_rows)],
                    sems[buf]))
            return hnds

        pend_g = fire(0, 0)
        pend_o = [None, None]
        for c in range(nsteps):
            b = c & 1
            nb = (c + 1) & 1
            if c + 1 < nsteps:
                if pend_o[nb] is not None:
                    pend_o[nb].wait()
                    pend_o[nb] = None
                nxt = fire(c + 1, nb)
            for h in pend_g:
                h.wait()
            pend_o[b] = pltpu.async_copy(
                rows[b], out_hbm.at[pl.ds(base + c * cb, cb)], osems[b])
            if c + 1 < nsteps:
                pend_g = nxt
        for b in range(2):
            if pend_o[b] is not None:
                pend_o[b].wait()

    return k(table, idx)


def _shift1(x, carry):
    r = pltpu.roll(x, shift=1, axis=0)
    li = lax.broadcasted_iota(jnp.int32, x.shape, 0)
    return jnp.where(li == 0, carry, r)


def _make_phase_body(offt, nt, first, last):
    def body(hs_ref, accin_ref, chin_ref, wprev_ref, wcur_ref,
             w1212_ref, b1212_ref, *rest):
        if last:
            (w13_ref, w21_ref, w22_ref, w23_ref, bs3_ref, b13_ref,
             b21_ref, b22_ref, b23_ref, gwp_ref,
             accout_ref, chout_ref, of_ref, os_ref, ow_ref, acc12, ch) = rest
        else:
            accout_ref, chout_ref, acc12, ch = rest
        t = pl.program_id(0)

        @pl.when(t == 0)
        def _init():
            acc12[...] = accin_ref[...]
            ch[...] = chin_ref[...].astype(jnp.bfloat16)

        hs = hs_ref[...]
        hs_b = hs.astype(jnp.bfloat16)
        hs_prev_b = _shift1(hs_b, ch[...])
        ch[...] = hs_b[-1:, :]

        ppe = jnp.dot(hs_prev_b, wprev_ref[...],
                      preferred_element_type=jnp.float32)
        p_prev = ppe[:, :D]
        q_prev = ppe[:, D:2 * D]
        e_prev = ppe[:, 2 * D:]
        qe = jnp.dot(hs_b, wcur_ref[...], preferred_element_type=jnp.float32)
        q = qe[:, :D]
        e_cur = qe[:, D:]
        el = jnp.maximum(e_cur, 0.2 * e_cur)
        el_prev = jnp.maximum(e_prev, 0.2 * e_prev)

        a = 1.0 / (1.0 + jnp.exp(el - el_prev))
        msg = q + a * (q_prev - q)

        rin = jnp.concatenate([hs + p_prev, msg],
                              axis=1).astype(jnp.bfloat16)
        r12 = jnp.maximum(
            jnp.dot(rin, w1212_ref[...], preferred_element_type=jnp.float32)
            + b1212_ref[...], 0.0)

        edge = None
        if first and nt == 1:
            edge = (t == 0)
        elif first:
            edge = (t == 0)
        if last:
            e2 = (t == nt - 1)
            edge = e2 if edge is None else (edge | e2)

        if edge is None:
            acc12[...] += jnp.sum(r12, axis=0, keepdims=True)
        else:
            @pl.when(jnp.logical_not(edge))
            def _mid():
                acc12[...] += jnp.sum(r12, axis=0, keepdims=True)

            @pl.when(edge)
            def _edge():
                row = (offt + t) * T + lax.broadcasted_iota(
                    jnp.int32, (T, 1), 0)
                r1m = jnp.where(row < L, r12[:, :D], 0.0)
                r2m = jnp.where((row >= 1) & (row <= L - 1), r12[:, D:], 0.0)
                acc12[:, :D] += jnp.sum(r1m, axis=0, keepdims=True)
                acc12[:, D:] += jnp.sum(r2m, axis=0, keepdims=True)

        @pl.when(t == nt - 1)
        def _fin():
            accout_ref[...] = acc12[...]
            chout_ref[...] = ch[...].astype(jnp.float32)
            if last:
                m1 = acc12[:, :D] * (1.0 / L)
                m2 = acc12[:, D:] * (1.0 / (L - 1))
                rep1 = jnp.dot(m1, w21_ref[...],
                               preferred_element_type=jnp.float32) + b21_ref[...]
                rep2 = jnp.dot(m2, w22_ref[...],
                               preferred_element_type=jnp.float32) + b22_ref[...]
                h3 = jnp.maximum(
                    jnp.dot(bs3_ref[...], w13_ref[...],
                            preferred_element_type=jnp.float32)
                    + b13_ref[...], 0.0)
                rep3 = jnp.dot(h3, w23_ref[...],
                               preferred_element_type=jnp.float32) + b23_ref[...]
                gwp = gwp_ref[...]
                ew = jnp.exp(gwp - jnp.max(gwp))
                wv = ew / jnp.sum(ew)
                fused = (wv[:, 0:1] * rep1 + wv[:, 1:2] * rep2
                         + wv[:, 2:3] * rep3)
                of_ref[...] = fused
                os_ref[0:1, :] = rep1
                os_ref[1:2, :] = rep2
                os_ref[2:3, :] = rep3
                ow_ref[...] = wv

    return body


def _dense_phase(hs_k, acc, chv, proj, ro, offt, nt, first, last):
    full = lambda shape: pl.BlockSpec(shape, lambda t: (0, 0))
    v = full((1, D))
    in_specs = [
        pl.BlockSpec((T, D), lambda t: (t, 0)),
        full((1, 2 * D)),
        v,
        full((D, 3 * D)),
        full((D, 2 * D)),
        full((2 * D, 2 * D)),
        full((1, 2 * D)),
    ]
    args = (hs_k, acc, chv) + proj
    out_specs = [full((1, 2 * D)), v]
    out_shape = [jax.ShapeDtypeStruct((1, 2 * D), jnp.float32),
                 jax.ShapeDtypeStruct((1, D), jnp.float32)]
    if last:
        in_specs += [
            full((D, D)),
            full((D, D)), full((D, D)), full((D, D)),
            v,
            v,
            v, v, v,
            v,
        ]
        args += ro
        out_specs += [v, full((3, D)), v]
        out_shape += [jax.ShapeDtypeStruct((1, D), jnp.float32),
                      jax.ShapeDtypeStruct((3, D), jnp.float32),
                      jax.ShapeDtypeStruct((1, D), jnp.float32)]
    return pl.pallas_call(
        _make_phase_body(offt, nt, first, last),
        grid=(nt,),
        in_specs=in_specs,
        out_specs=out_specs,
        out_shape=out_shape,
        scratch_shapes=[pltpu.VMEM((1, 2 * D), jnp.float32),
                        pltpu.VMEM((1, D), jnp.bfloat16)],
        compiler_params=pltpu.CompilerParams(
            dimension_semantics=("arbitrary",)),
    )(*args)


def kernel(items, params):
    idxp = jnp.pad(items, (0, LP - L))

    gidx = jnp.arange(D, dtype=jnp.int32) // DH
    bones = (gidx[:, None] == gidx[None, :]).astype(jnp.float32)
    wel = jnp.dot(params['W_up1'] * params['al_up1'].reshape(1, D), bones)
    wprev = jnp.concatenate([params['W_seq1'], params['W_up1'], wel],
                            axis=1).astype(jnp.bfloat16)
    wcur = jnp.concatenate([params['W_up1'], wel],
                           axis=1).astype(jnp.bfloat16)
    z = jnp.zeros((D, D), jnp.float32)
    w1212 = jnp.block([[params['ro1_W1'], z],
                       [z, params['ro2_W1']]]).astype(jnp.bfloat16)
    bsum1 = params['b_seq1'] + params['b_down1']
    bsum2 = params['b_seq2'] + params['b_down2'] + params['b_up1']
    b1212 = jnp.concatenate(
        [jnp.dot(bsum1, params['ro1_W1']) + params['ro1_b1'],
         jnp.dot(bsum2, params['ro2_W1']) + params['ro2_b1']]).reshape(1, 2 * D)
    r = lambda x: x.reshape(1, D)
    proj = (wprev, wcur, w1212, b1212)
    ro = (params['ro3_W1'],
          params['ro1_W2'], params['ro2_W2'], params['ro3_W2'],
          r(params['b_seq3'] + params['b_up2']),
          r(params['ro3_b1']),
          r(params['ro1_b2']), r(params['ro2_b2']), r(params['ro3_b2']),
          jnp.concatenate([params['gw'],
                           jnp.full((D - 3,), -jnp.inf, jnp.float32)]
                          ).reshape(1, D))

    segs = []
    offt = 0
    for nt in PHASES:
        segs.append((offt, nt))
        offt += nt
    hs_parts = [_sc_gather(params['emb'],
                           idxp if (o + n) * T > L else items, o, n)
                for o, n in segs]

    acc = jnp.zeros((1, 2 * D), jnp.float32)
    chv = jnp.zeros((1, D), jnp.float32)
    for k_i, (o, n) in enumerate(segs):
        res = _dense_phase(hs_parts[k_i], acc, chv, proj, ro, o, n,
                           first=(k_i == 0), last=(k_i == len(segs) - 1))
        acc, chv = res[0], res[1]
    of, os_, ow = res[2], res[3], res[4]
    return of.reshape(D), os_, ow[0, :3]


# speedup vs baseline: 702.3948x; 1.0177x over previous
import functools

import jax
import jax.numpy as jnp
from jax import lax
from jax.experimental import pallas as pl
from jax.experimental.pallas import tpu as pltpu
from jax.experimental.pallas import tpu_sc as plsc

L = 50000
D = 128
DH = 16
NW = 32
LP = 50176
T = 7168
G = LP // T
PHASES = (1, 2, 2, 2)


def _pick_cb(bpw):
    for d in range(min(bpw, 256), 0, -16):
        if bpw % d == 0 and (d // 2) % 8 == 0 and d // 2 <= 128:
            return d
    raise ValueError(bpw)


def _sc_gather(table, idx, offt, nt):
    mesh = plsc.VectorSubcoreMesh(core_axis_name="c", subcore_axis_name="s")
    bpw = nt * T // NW
    cb = _pick_cb(bpw)
    ch_rows = cb // 2
    nsteps = bpw // cb
    off = offt * T

    @functools.partial(
        pl.kernel,
        out_type=jax.ShapeDtypeStruct((nt * T, D), jnp.float32),
        mesh=mesh,
        scratch_types=[
            pltpu.VMEM((bpw,), jnp.int32),
            pltpu.VMEM((cb, D), jnp.float32),
            pltpu.VMEM((cb, D), jnp.float32),
            pltpu.SemaphoreType.DMA,
            pltpu.SemaphoreType.DMA,
            pltpu.SemaphoreType.DMA,
            pltpu.SemaphoreType.DMA,
        ],
    )
    def k(table_hbm, idx_hbm, out_hbm, idx_v, rows0, rows1,
          sem0, sem1, osem0, osem1):
        wid = lax.axis_index("s") * 2 + lax.axis_index("c")
        base = wid * bpw
        pltpu.sync_copy(idx_hbm.at[pl.ds(off + base, bpw)], idx_v)
        rows = (rows0, rows1)
        sems = (sem0, sem1)
        osems = (osem0, osem1)

        def fire(step, buf):
            hnds = []
            for h in range(2):
                hnds.append(pltpu.async_copy(
                    table_hbm.at[idx_v.at[pl.ds(step * cb + h * ch_rows,
                                                ch_rows)]],
                    rows[buf].at[pl.ds(h * ch_rows, ch_rows)],
                    sems[buf]))
            return hnds

        pend_g = fire(0, 0)
        pend_o = [None, None]
        for c in range(nsteps):
            b = c & 1
            nb = (c + 1) & 1
            if c + 1 < nsteps:
                if pend_o[nb] is not None:
                    pend_o[nb].wait()
                    pend_o[nb] = None
                nxt = fire(c + 1, nb)
            for h in pend_g:
                h.wait()
            pend_o[b] = pltpu.async_copy(
                rows[b], out_hbm.at[pl.ds(base + c * cb, cb)], osems[b])
            if c + 1 < nsteps:
                pend_g = nxt
        for b in range(2):
            if pend_o[b] is not None:
                pend_o[b].wait()

    return k(table, idx)


def _shift1(x, carry):
    r = pltpu.roll(x, shift=1, axis=0)
    li = lax.broadcasted_iota(jnp.int32, x.shape, 0)
    return jnp.where(li == 0, carry, r)


def _make_phase_body(offt, nt, first, last):
    def body(hs_ref, accin_ref, chin_ref, wprev_ref, wcur_ref,
             w1212_ref, b1212_ref, *rest):
        if last:
            (w13_ref, w21_ref, w22_ref, w23_ref, bs3_ref, b13_ref,
             b21_ref, b22_ref, b23_ref, gwp_ref,
             accout_ref, chout_ref, of_ref, os_ref, ow_ref, acc12, ch) = rest
        else:
            accout_ref, chout_ref, acc12, ch = rest
        t = pl.program_id(0)

        @pl.when(t == 0)
        def _init():
            acc12[...] = accin_ref[...]
            ch[...] = chin_ref[...].astype(jnp.bfloat16)

        hs = hs_ref[...]
        hs_b = hs.astype(jnp.bfloat16)
        hs_prev_b = _shift1(hs_b, ch[...])
        ch[...] = hs_b[-1:, :]

        ppe = jnp.dot(hs_prev_b, wprev_ref[...],
                      preferred_element_type=jnp.float32)
        p_prev = ppe[:, :D]
        q_prev = ppe[:, D:2 * D]
        e_prev = ppe[:, 2 * D:]
        qe = jnp.dot(hs_b, wcur_ref[...], preferred_element_type=jnp.float32)
        q = qe[:, :D]
        e_cur = qe[:, D:]
        el = jnp.maximum(e_cur, 0.2 * e_cur)
        el_prev = jnp.maximum(e_prev, 0.2 * e_prev)

        a = 1.0 / (1.0 + jnp.exp(el - el_prev))
        msg = q + a * (q_prev - q)

        rin = jnp.concatenate([hs + p_prev, msg],
                              axis=1).astype(jnp.bfloat16)
        r12 = jnp.maximum(
            jnp.dot(rin, w1212_ref[...], preferred_element_type=jnp.float32)
            + b1212_ref[...], 0.0)

        edge = None
        if first and nt == 1:
            edge = (t == 0)
        elif first:
            edge = (t == 0)
        if last:
            e2 = (t == nt - 1)
            edge = e2 if edge is None else (edge | e2)

        if edge is None:
            acc12[...] += jnp.sum(r12, axis=0, keepdims=True)
        else:
            @pl.when(jnp.logical_not(edge))
            def _mid():
                acc12[...] += jnp.sum(r12, axis=0, keepdims=True)

            @pl.when(edge)
            def _edge():
                row = (offt + t) * T + lax.broadcasted_iota(
                    jnp.int32, (T, 1), 0)
                r1m = jnp.where(row < L, r12[:, :D], 0.0)
                r2m = jnp.where((row >= 1) & (row <= L - 1), r12[:, D:], 0.0)
                acc12[:, :D] += jnp.sum(r1m, axis=0, keepdims=True)
                acc12[:, D:] += jnp.sum(r2m, axis=0, keepdims=True)

        @pl.when(t == nt - 1)
        def _fin():
            accout_ref[...] = acc12[...]
            chout_ref[...] = ch[...].astype(jnp.float32)
            if last:
                m1 = acc12[:, :D] * (1.0 / L)
                m2 = acc12[:, D:] * (1.0 / (L - 1))
                rep1 = jnp.dot(m1, w21_ref[...],
                               preferred_element_type=jnp.float32) + b21_ref[...]
                rep2 = jnp.dot(m2, w22_ref[...],
                               preferred_element_type=jnp.float32) + b22_ref[...]
                h3 = jnp.maximum(
                    jnp.dot(bs3_ref[...], w13_ref[...],
                            preferred_element_type=jnp.float32)
                    + b13_ref[...], 0.0)
                rep3 = jnp.dot(h3, w23_ref[...],
                               preferred_element_type=jnp.float32) + b23_ref[...]
                gwp = gwp_ref[...]
                ew = jnp.exp(gwp - jnp.max(gwp))
                wv = ew / jnp.sum(ew)
                fused = (wv[:, 0:1] * rep1 + wv[:, 1:2] * rep2
                         + wv[:, 2:3] * rep3)
                of_ref[...] = fused
                os_ref[0:1, :] = rep1
                os_ref[1:2, :] = rep2
                os_ref[2:3, :] = rep3
                ow_ref[...] = wv

    return body


def _dense_phase(hs_k, acc, chv, proj, ro, offt, nt, first, last):
    full = lambda shape: pl.BlockSpec(shape, lambda t: (0, 0))
    v = full((1, D))
    in_specs = [
        pl.BlockSpec((T, D), lambda t: (t, 0)),
        full((1, 2 * D)),
        v,
        full((D, 3 * D)),
        full((D, 2 * D)),
        full((2 * D, 2 * D)),
        full((1, 2 * D)),
    ]
    args = (hs_k, acc, chv) + proj
    out_specs = [full((1, 2 * D)), v]
    out_shape = [jax.ShapeDtypeStruct((1, 2 * D), jnp.float32),
                 jax.ShapeDtypeStruct((1, D), jnp.float32)]
    if last:
        in_specs += [
            full((D, D)),
            full((D, D)), full((D, D)), full((D, D)),
            v,
            v,
            v, v, v,
            v,
        ]
        args += ro
        out_specs += [v, full((3, D)), v]
        out_shape += [jax.ShapeDtypeStruct((1, D), jnp.float32),
                      jax.ShapeDtypeStruct((3, D), jnp.float32),
                      jax.ShapeDtypeStruct((1, D), jnp.float32)]
    return pl.pallas_call(
        _make_phase_body(offt, nt, first, last),
        grid=(nt,),
        in_specs=in_specs,
        out_specs=out_specs,
        out_shape=out_shape,
        scratch_shapes=[pltpu.VMEM((1, 2 * D), jnp.float32),
                        pltpu.VMEM((1, D), jnp.bfloat16)],
        compiler_params=pltpu.CompilerParams(
            dimension_semantics=("arbitrary",)),
    )(*args)


def kernel(items, params):
    idxp = jnp.pad(items, (0, LP - L))

    gidx = jnp.arange(D, dtype=jnp.int32) // DH
    bones = (gidx[:, None] == gidx[None, :]).astype(jnp.float32)
    wel = jnp.dot(params['W_up1'] * params['al_up1'].reshape(1, D), bones)
    wprev = jnp.concatenate([params['W_seq1'], params['W_up1'], wel],
                            axis=1).astype(jnp.bfloat16)
    wcur = jnp.concatenate([params['W_up1'], wel],
                           axis=1).astype(jnp.bfloat16)
    z = jnp.zeros((D, D), jnp.float32)
    w1212 = jnp.block([[params['ro1_W1'], z],
                       [z, params['ro2_W1']]]).astype(jnp.bfloat16)
    bsum1 = params['b_seq1'] + params['b_down1']
    bsum2 = params['b_seq2'] + params['b_down2'] + params['b_up1']
    b1212 = jnp.concatenate(
        [jnp.dot(bsum1, params['ro1_W1']) + params['ro1_b1'],
         jnp.dot(bsum2, params['ro2_W1']) + params['ro2_b1']]).reshape(1, 2 * D)
    r = lambda x: x.reshape(1, D)
    proj = (wprev, wcur, w1212, b1212)
    ro = (params['ro3_W1'],
          params['ro1_W2'], params['ro2_W2'], params['ro3_W2'],
          r(params['b_seq3'] + params['b_up2']),
          r(params['ro3_b1']),
          r(params['ro1_b2']), r(params['ro2_b2']), r(params['ro3_b2']),
          jnp.concatenate([params['gw'],
                           jnp.full((D - 3,), -jnp.inf, jnp.float32)]
                          ).reshape(1, D))

    segs = []
    offt = 0
    for nt in PHASES:
        segs.append((offt, nt))
        offt += nt
    hs_parts = [_sc_gather(params['emb'],
                           idxp if (o + n) * T > L else items, o, n)
                for o, n in segs]

    acc = jnp.zeros((1, 2 * D), jnp.float32)
    chv = jnp.zeros((1, D), jnp.float32)
    for k_i, (o, n) in enumerate(segs):
        res = _dense_phase(hs_parts[k_i], acc, chv, proj, ro, o, n,
                           first=(k_i == 0), last=(k_i == len(segs) - 1))
        acc, chv = res[0], res[1]
    of, os_, ow = res[2], res[3], res[4]
    return of.reshape(D), os_, ow[0, :3]
